# Initial kernel scaffold; baseline (speedup 1.0000x reference)
import jax
import jax.numpy as jnp
from jax.experimental import pallas as pl


def kernel(x, edge_index, expander_edge_index, batch, W1, b1, W2, b2, W3, b3):
    raise NotImplementedError("write your pallas kernel here")


# speedup vs baseline: 4.2922x; 4.2922x over previous
import functools

import jax
import jax.numpy as jnp
from jax import lax
from jax.experimental import pallas as pl
from jax.experimental.pallas import tpu as pltpu
from jax.experimental.pallas import tpu_sc as plsc

N = 10000
D = 128
E = 320000
G = 64
HALF = 64

NC = 2
NS = 16
K = 128
NCHUNK = 158
EPT = NCHUNK * K
RPT = N // NS
ACC_ROWS = N + 16
PQ = 5
PK = RPT // PQ

_mesh = plsc.VectorSubcoreMesh(core_axis_name="c", subcore_axis_name="s")
_sc_params = pltpu.CompilerParams(use_tc_tiling_on_sc=False)


def _agg_impl(do_pool, h_hbm, src_hbm, dst_hbm, *rest):
    if do_pool:
        (batch_hbm, zero_hbm, pool_out,
         acc, idx_s, idx_d, rows, stage, pooled, bidx, sem) = rest
    else:
        (out_hbm, acc, idx_s, idx_d, rows, stage, sem) = rest
    c = lax.axis_index("c")
    s = lax.axis_index("s")
    base = s * RPT

    pltpu.sync_copy(src_hbm.at[c, s], idx_s)
    pltpu.sync_copy(dst_hbm.at[s], idx_d)

    for q in range(PQ):
        pltpu.sync_copy(h_hbm.at[pl.ds(c * N + base + q * PK, PK)], stage)
        pltpu.sync_copy(stage, acc.at[pl.ds(base + q * PK, PK)])
    plsc.subcore_barrier()

    def body(j, carry):
        pltpu.async_copy(h_hbm.at[idx_s.at[j]], rows.at[0], sem).wait()
        pltpu.sync_copy(rows.at[0], acc.at[idx_d.at[j]], add=True)
        return carry

    lax.fori_loop(0, NCHUNK, body, 0)
    plsc.subcore_barrier()

    if not do_pool:
        for q in range(PQ):
            pltpu.sync_copy(acc.at[pl.ds(base + q * PK, PK)], stage)
            pltpu.sync_copy(stage, out_hbm.at[pl.ds(c * N + base + q * PK, PK)])
        return

    @pl.when(s == 0)
    def _():
        pltpu.sync_copy(zero_hbm, pooled)
    plsc.subcore_barrier()
    pltpu.sync_copy(batch_hbm.at[s], bidx)
    for q in range(PQ):
        pltpu.sync_copy(acc.at[pl.ds(base + q * PK, PK)], stage)
        pltpu.sync_copy(stage, pooled.at[bidx.at[q]], add=True)
    plsc.subcore_barrier()

    @pl.when(s == 0)
    def _():
        pltpu.sync_copy(pooled, stage.at[pl.ds(0, G)])
        pltpu.sync_copy(stage.at[pl.ds(0, G)], pool_out.at[c])


_agg = functools.partial(
    pl.kernel,
    functools.partial(_agg_impl, False),
    out_type=jax.ShapeDtypeStruct((2 * N, HALF), jnp.float32),
    mesh=_mesh,
    scratch_types=[
        pltpu.VMEM_SHARED((ACC_ROWS, HALF), jnp.float32),
        pltpu.VMEM((NCHUNK, K), jnp.int32),
        pltpu.VMEM((NCHUNK, K), jnp.int32),
        pltpu.VMEM((2, K, HALF), jnp.float32),
        pltpu.VMEM((PK, HALF), jnp.float32),
        pltpu.SemaphoreType.DMA,
    ],
    compiler_params=_sc_params,
)()

_agg_pool = functools.partial(
    pl.kernel,
    functools.partial(_agg_impl, True),
    out_type=jax.ShapeDtypeStruct((NC, G, HALF), jnp.float32),
    mesh=_mesh,
    scratch_types=[
        pltpu.VMEM_SHARED((ACC_ROWS, HALF), jnp.float32),
        pltpu.VMEM((NCHUNK, K), jnp.int32),
        pltpu.VMEM((NCHUNK, K), jnp.int32),
        pltpu.VMEM((2, K, HALF), jnp.float32),
        pltpu.VMEM((PK, HALF), jnp.float32),
        pltpu.VMEM_SHARED((G, HALF), jnp.float32),
        pltpu.VMEM((PQ, PK), jnp.int32),
        pltpu.SemaphoreType.DMA,
    ],
    compiler_params=_sc_params,
)()


BLK = 2000


def _mm_body(h0_ref, h1_ref, wt_ref, b_ref, o_ref):
    h0 = h0_ref[...]
    h1 = h1_ref[...]
    wt = wt_ref[0]
    acc = lax.dot_general(h0, wt[:HALF], (((1,), (0,)), ((), ())),
                          preferred_element_type=jnp.float32)
    acc += lax.dot_general(h1, wt[HALF:], (((1,), (0,)), ((), ())),
                           preferred_element_type=jnp.float32)
    o_ref[...] = jnp.maximum(acc + b_ref[0], 0.0)


_mm = pl.pallas_call(
    _mm_body,
    grid=(2, N // BLK),
    in_specs=[
        pl.BlockSpec((BLK, HALF), lambda half, i: (i, 0)),
        pl.BlockSpec((BLK, HALF), lambda half, i: (N // BLK + i, 0)),
        pl.BlockSpec((1, D, HALF), lambda half, i: (half, 0, 0)),
        pl.BlockSpec((1, 1, HALF), lambda half, i: (half, 0, 0)),
    ],
    out_specs=pl.BlockSpec((BLK, HALF), lambda half, i: (half * (N // BLK) + i, 0)),
    out_shape=jax.ShapeDtypeStruct((2 * N, HALF), jnp.float32),
)


def _prep_edges(ei):
    src, dst = ei[0], ei[1]
    pad = NS * NCHUNK * K - E
    src = jnp.concatenate([src, jnp.zeros((pad,), jnp.int32)])
    dst = jnp.concatenate([dst, jnp.full((pad,), N, jnp.int32)])
    src = src.reshape(NS, NCHUNK, K)
    src2 = src[None] + (jnp.arange(NC, dtype=jnp.int32) * N)[:, None, None, None]
    return src2, dst.reshape(NS, NCHUNK, K)


def kernel(x, edge_index, expander_edge_index, batch, W1, b1, W2, b2, W3, b3):
    h = jnp.concatenate([x[:, :HALF], x[:, HALF:]], axis=0)
    src_e, dst_e = _prep_edges(edge_index)
    src_x, dst_x = _prep_edges(expander_edge_index)
    batch_i = batch.reshape(NS, PQ, PK)
    zero = jnp.zeros((G, HALF), jnp.float32)
    for li, (W, b) in enumerate(((W1, b1), (W2, b2), (W3, b3))):
        h = _agg(h, src_e, dst_e)
        wt = W.T.reshape(1, D, D)
        wt = jnp.concatenate([wt[:, :, :HALF], wt[:, :, HALF:]], axis=0)
        h = _mm(h, h, wt, b.reshape(2, 1, HALF))
        if li < 2:
            h = _agg(h, src_x, dst_x)
        else:
            pooled = _agg_pool(h, src_x, dst_x, batch_i, zero)
    return jnp.concatenate([pooled[0], pooled[1]], axis=1).reshape(-1)
